# baseline (speedup 1.0000x reference)
import functools

import jax
import jax.numpy as jnp
from jax.experimental import pallas as pl
from jax.experimental.pallas import tpu as pltpu

B, L, H, K, D = 64, 1024, 32, 100000, 256
BLK = 2048
NK = (K + BLK - 1) // BLK
KP = NK * BLK

_NEG_INF = float("-inf")


def _fused_kernel(x_ref, w1_ref, b1p_ref, w2_ref, b2_ref,
                  logits_ref, probs_ref, actions_ref,
                  h_s, l_s, m_s, s_s, am_s):
    i = pl.program_id(0)

    @pl.when(i == 0)
    def _init():
        h = jnp.maximum(
            jnp.dot(x_ref[...], w1_ref[...], preferred_element_type=jnp.float32)
            + b1p_ref[...], 0.0)
        h_s[...] = h
        m_s[...] = jnp.full((B, 1), _NEG_INF, jnp.float32)
        s_s[...] = jnp.zeros((B, 1), jnp.float32)
        am_s[...] = jnp.zeros((B, 1), jnp.int32)

    @pl.when(i < NK)
    def _phase_a():
        k = i
        logits = (jnp.dot(h_s[...], w2_ref[...], preferred_element_type=jnp.float32)
                  + b2_ref[...])
        cols = k * BLK + jax.lax.broadcasted_iota(jnp.int32, (B, BLK), 1)
        logits = jnp.where(cols < K, logits, _NEG_INF)
        logits_ref[...] = logits
        l_s[:, pl.ds(k * BLK, BLK)] = logits

        bm = jnp.max(logits, axis=1, keepdims=True)
        cand = jnp.where(logits == bm, cols, K)
        bam = jnp.min(cand, axis=1, keepdims=True)

        m_old = m_s[...]
        better = bm > m_old
        am_s[...] = jnp.where(better, bam, am_s[...])
        m_new = jnp.maximum(m_old, bm)
        s_s[...] = (s_s[...] * jnp.exp(m_old - m_new)
                    + jnp.sum(jnp.exp(logits - m_new), axis=1, keepdims=True))
        m_s[...] = m_new

    @pl.when(i >= NK)
    def _phase_b():
        k2 = i - NK
        blk_l = l_s[:, pl.ds(k2 * BLK, BLK)]
        probs_ref[...] = jnp.exp(blk_l - m_s[...]) / s_s[...]

        @pl.when(i == 2 * NK - 1)
        def _final():
            actions_ref[...] = am_s[...].astype(jnp.float32)


@functools.partial(jax.jit, static_argnames=())
def _forward(x, W1, b1p, W2, b2row):
    grid = (2 * NK,)
    out_logits, out_probs, out_actions = pl.pallas_call(
        _fused_kernel,
        grid=grid,
        in_specs=[
            pl.BlockSpec((B, L + 2 * H), lambda i: (0, 0)),
            pl.BlockSpec((L + 2 * H, D), lambda i: (0, 0)),
            pl.BlockSpec((1, D), lambda i: (0, 0)),
            pl.BlockSpec((D, BLK), lambda i: (0, jnp.minimum(i, NK - 1))),
            pl.BlockSpec((1, BLK), lambda i: (0, jnp.minimum(i, NK - 1))),
        ],
        out_specs=[
            pl.BlockSpec((B, BLK), lambda i: (0, jnp.minimum(i, NK - 1))),
            pl.BlockSpec((B, BLK), lambda i: (0, jnp.maximum(i - NK, 0))),
            pl.BlockSpec((B, 1), lambda i: (0, 0)),
        ],
        out_shape=[
            jax.ShapeDtypeStruct((B, K), jnp.float32),
            jax.ShapeDtypeStruct((B, K), jnp.float32),
            jax.ShapeDtypeStruct((B, 1), jnp.float32),
        ],
        scratch_shapes=[
            pltpu.VMEM((B, D), jnp.float32),
            pltpu.VMEM((B, KP), jnp.float32),
            pltpu.VMEM((B, 1), jnp.float32),
            pltpu.VMEM((B, 1), jnp.float32),
            pltpu.VMEM((B, 1), jnp.int32),
        ],
    )(x, W1, b1p, W2, b2row)
    return out_logits, out_probs, out_actions


def kernel(context, forecast, forecast_mask, step, W1, b1, W2, b2, pos_emb):
    m = forecast_mask.astype(jnp.float32)
    x = jnp.concatenate([context, forecast * m, m], axis=-1)
    b1p = (b1 + pos_emb[step]).reshape(1, D)
    b2row = b2.reshape(1, K)
    logits, probs, actions = _forward(x, W1, b1p, W2, b2row)
    return (actions.reshape(B), probs, logits)


# speedup vs baseline: 1.1285x; 1.1285x over previous
import jax
import jax.numpy as jnp
from jax.experimental import pallas as pl
from jax.experimental.pallas import tpu as pltpu

B, L, H, K, D = 64, 1024, 32, 100000, 256
BLK = 4096
NK = (K + BLK - 1) // BLK
KP = NK * BLK
NKP = 128

_NEG_INF = float("-inf")


def _fused_kernel(x_ref, w1_ref, b1p_ref, w2_ref, b2_ref,
                  logits_ref, probs_ref, actions_ref,
                  h_s, e_s, bm_s, m_s, s_s, am_s):
    i = pl.program_id(0)

    @pl.when(i == 0)
    def _init():
        h = jnp.maximum(
            jnp.dot(x_ref[...], w1_ref[...], preferred_element_type=jnp.float32)
            + b1p_ref[...], 0.0)
        h_s[...] = h
        m_s[...] = jnp.full((B, 1), _NEG_INF, jnp.float32)
        s_s[...] = jnp.zeros((B, 1), jnp.float32)
        am_s[...] = jnp.zeros((B, 1), jnp.int32)

    @pl.when(i < NK)
    def _phase_a():
        k = i
        logits = (jnp.dot(h_s[...], w2_ref[...], preferred_element_type=jnp.float32)
                  + b2_ref[...])
        cols = k * BLK + jax.lax.broadcasted_iota(jnp.int32, (B, BLK), 1)
        logits = jax.lax.cond(
            k == NK - 1,
            lambda l: jnp.where(cols < K, l, _NEG_INF),
            lambda l: l,
            logits)
        logits_ref[...] = logits

        bm = jnp.max(logits, axis=1, keepdims=True)
        e = jnp.exp(logits - bm)
        e_s[:, pl.ds(k * BLK, BLK)] = e
        lane = jax.lax.broadcasted_iota(jnp.int32, (B, NKP), 1)
        bm_s[...] = jnp.where(lane == k, jnp.broadcast_to(bm, (B, NKP)),
                              bm_s[...])

        cand = jnp.where(logits == bm, cols, K)
        bam = jnp.min(cand, axis=1, keepdims=True)

        m_old = m_s[...]
        better = bm > m_old
        am_s[...] = jnp.where(better, bam, am_s[...])
        m_new = jnp.maximum(m_old, bm)
        s_s[...] = (s_s[...] * jnp.exp(m_old - m_new)
                    + jnp.sum(e, axis=1, keepdims=True) * jnp.exp(bm - m_new))
        m_s[...] = m_new

    @pl.when(i >= NK)
    def _phase_b():
        k2 = i - NK
        lane = jax.lax.broadcasted_iota(jnp.int32, (B, NKP), 1)
        bm_k = jnp.sum(jnp.where(lane == k2, bm_s[...], 0.0), axis=1,
                       keepdims=True)
        scale = jnp.exp(bm_k - m_s[...]) / s_s[...]
        probs_ref[...] = e_s[:, pl.ds(k2 * BLK, BLK)] * scale

        @pl.when(i == 2 * NK - 1)
        def _final():
            actions_ref[...] = am_s[...].astype(jnp.float32)


def _forward(x, W1, b1p, W2, b2row):
    grid = (2 * NK,)
    out_logits, out_probs, out_actions = pl.pallas_call(
        _fused_kernel,
        grid=grid,
        in_specs=[
            pl.BlockSpec((B, L + 2 * H), lambda i: (0, 0)),
            pl.BlockSpec((L + 2 * H, D), lambda i: (0, 0)),
            pl.BlockSpec((1, D), lambda i: (0, 0)),
            pl.BlockSpec((D, BLK), lambda i: (0, jnp.minimum(i, NK - 1))),
            pl.BlockSpec((1, BLK), lambda i: (0, jnp.minimum(i, NK - 1))),
        ],
        out_specs=[
            pl.BlockSpec((B, BLK), lambda i: (0, jnp.minimum(i, NK - 1))),
            pl.BlockSpec((B, BLK), lambda i: (0, jnp.maximum(i - NK, 0))),
            pl.BlockSpec((B, 1), lambda i: (0, 0)),
        ],
        out_shape=[
            jax.ShapeDtypeStruct((B, K), jnp.float32),
            jax.ShapeDtypeStruct((B, K), jnp.float32),
            jax.ShapeDtypeStruct((B, 1), jnp.float32),
        ],
        scratch_shapes=[
            pltpu.VMEM((B, D), jnp.float32),
            pltpu.VMEM((B, KP), jnp.float32),
            pltpu.VMEM((B, NKP), jnp.float32),
            pltpu.VMEM((B, 1), jnp.float32),
            pltpu.VMEM((B, 1), jnp.float32),
            pltpu.VMEM((B, 1), jnp.int32),
        ],
    )(x, W1, b1p, W2, b2row)
    return out_logits, out_probs, out_actions


def kernel(context, forecast, forecast_mask, step, W1, b1, W2, b2, pos_emb):
    m = forecast_mask.astype(jnp.float32)
    x = jnp.concatenate([context, forecast * m, m], axis=-1)
    b1p = (b1 + pos_emb[step]).reshape(1, D)
    b2row = b2.reshape(1, K)
    logits, probs, actions = _forward(x, W1, b1p, W2, b2row)
    return (actions.reshape(B), probs, logits)
